# Initial kernel scaffold; baseline (speedup 1.0000x reference)
import jax
import jax.numpy as jnp
from jax.experimental import pallas as pl


def kernel(predicted_locs, predicted_scores, boxes, labels, priors):
    raise NotImplementedError("write your pallas kernel here")


# speedup vs baseline: 10.7109x; 10.7109x over previous
import jax
import jax.numpy as jnp
from jax import lax
from jax.experimental import pallas as pl

_THRESHOLD = 0.5


def _match_body(locs_ref, scores_ref, boxes_ref, labels_ref, priors_ref,
                neg_ref, stats_ref):
    O = boxes_ref.shape[1]
    C = scores_ref.shape[1]
    P = priors_ref.shape[1]

    pcx = priors_ref[0:1, :]
    pcy = priors_ref[1:2, :]
    pw = priors_ref[2:3, :]
    ph = priors_ref[3:4, :]
    px1 = pcx - pw * 0.5
    py1 = pcy - ph * 0.5
    px2 = pcx + pw * 0.5
    py2 = pcy + ph * 0.5

    bx1 = boxes_ref[0, :, 0:1]
    by1 = boxes_ref[0, :, 1:2]
    bx2 = boxes_ref[0, :, 2:3]
    by2 = boxes_ref[0, :, 3:4]

    iw = jnp.maximum(jnp.minimum(bx2, px2) - jnp.maximum(bx1, px1), 0.0)
    ih = jnp.maximum(jnp.minimum(by2, py2) - jnp.maximum(by1, py1), 0.0)
    inter = iw * ih
    area_a = (bx2 - bx1) * (by2 - by1)
    area_b = (px2 - px1) * (py2 - py1)
    ov = inter / (area_a + area_b - inter)

    ofep = jnp.max(ov, axis=0, keepdims=True)
    oidx = lax.broadcasted_iota(jnp.int32, (O, P), 0).astype(jnp.float32)
    obj = jnp.min(jnp.where(ov == ofep, oidx, float(O)), axis=0, keepdims=True)

    rowmax = jnp.max(ov, axis=1, keepdims=True)
    pidx = lax.broadcasted_iota(jnp.int32, (O, P), 1).astype(jnp.float32)
    pfeo = jnp.min(jnp.where(ov == rowmax, pidx, float(P)), axis=1,
                   keepdims=True)
    forced = jnp.where(pidx == pfeo, 1.0, 0.0)
    any_forced = jnp.max(forced, axis=0, keepdims=True)
    last_obj = jnp.max(jnp.where(forced > 0.0, oidx, -1.0), axis=0,
                       keepdims=True)
    obj = jnp.where(any_forced > 0.0, last_obj, obj)
    ofep = jnp.where(any_forced > 0.0, 1.0, ofep)

    oh = oidx == obj

    def gather(src):
        return jnp.sum(jnp.where(oh, src, 0.0), axis=0, keepdims=True)

    gx1 = gather(bx1)
    gy1 = gather(by1)
    gx2 = gather(bx2)
    gy2 = gather(by2)
    glab = gather(labels_ref[0])

    lbl = jnp.where(ofep < _THRESHOLD, 0.0, glab)
    posf = jnp.where(lbl != 0.0, 1.0, 0.0)

    cx = (gx1 + gx2) / 2.0
    cy = (gy1 + gy2) / 2.0
    w = gx2 - gx1
    h = gy2 - gy1
    tgx = (cx - pcx) / (pw / 10.0)
    tgy = (cy - pcy) / (ph / 10.0)
    tgw = jnp.log(w / pw) * 5.0
    tgh = jnp.log(h / ph) * 5.0

    loc_sum = jnp.float32(0.0)
    for c, t in enumerate((tgx, tgy, tgw, tgh)):
        pd = locs_ref[0, c:c + 1, :]
        pd = jnp.where(jnp.isnan(pd), 0.0, pd)
        t = jnp.where(jnp.isnan(t), 0.0, t)
        d = pd - t
        ad = jnp.abs(d)
        e = jnp.where(ad < 1.0, 0.5 * d * d, ad - 0.5)
        loc_sum = loc_sum + jnp.sum(jnp.where(posf > 0.0, e, 0.0))

    n_pos = jnp.sum(posf)

    x = scores_ref[0]
    m = jnp.max(x, axis=0, keepdims=True)
    sumexp = jnp.sum(jnp.exp(x - m), axis=0, keepdims=True)
    lse = m + jnp.log(sumexp)
    cidx = lax.broadcasted_iota(jnp.int32, (C, P), 0).astype(jnp.float32)
    xc = jnp.sum(jnp.where(cidx == lbl, x, 0.0), axis=0, keepdims=True)
    conf_all = lse - xc
    conf_pos = jnp.sum(jnp.where(posf > 0.0, conf_all, 0.0))
    neg_ref[0] = jnp.where(posf > 0.0, 0.0, conf_all)

    lane = lax.broadcasted_iota(jnp.int32, (1, 128), 1).astype(jnp.float32)
    stats_ref[0] = jnp.where(lane == 0.0, loc_sum,
                   jnp.where(lane == 1.0, n_pos,
                   jnp.where(lane == 2.0, conf_pos, 0.0)))


def _select_body(neg_ref, stats_ref, out_ref):
    B, P = neg_ref.shape
    x = neg_ref[...]
    st = stats_ref[...]
    loc_sum = st[:, 0:1]
    n_pos = st[:, 1:2]
    conf_pos = st[:, 2:3]
    k = jnp.minimum(n_pos * 3.0, float(P))

    i = lax.bitcast_convert_type(x, jnp.int32)
    y = jnp.where(i < 0, i ^ jnp.int32(0x7FFFFFFF), i)

    p = jnp.full((B, 1), -2147483648, dtype=jnp.int32)
    cnt0 = jnp.sum(jnp.where(y >= 0, 1.0, 0.0), axis=1, keepdims=True)
    p = jnp.where(cnt0 >= k, jnp.int32(0), p)
    for bit in range(30, -1, -1):
        cand = p + jnp.int32(1 << bit)
        cnt = jnp.sum(jnp.where(y >= cand, 1.0, 0.0), axis=1, keepdims=True)
        p = jnp.where(cnt >= k, cand, p)

    gt = y > p
    cnt_gt = jnp.sum(jnp.where(gt, 1.0, 0.0), axis=1, keepdims=True)
    sum_gt = jnp.sum(jnp.where(gt, x, 0.0), axis=1, keepdims=True)
    it = jnp.where(p < 0, p ^ jnp.int32(0x7FFFFFFF), p)
    t = lax.bitcast_convert_type(it, jnp.float32)
    rem = k - cnt_gt
    hard = sum_gt + jnp.where(rem > 0.0, rem * t, 0.0)

    npos_tot = jnp.sum(n_pos, keepdims=True)
    conf_loss = (jnp.sum(hard, keepdims=True)
                 + jnp.sum(conf_pos, keepdims=True)) / npos_tot
    loc_loss = jnp.sum(loc_sum, keepdims=True) / (npos_tot * 4.0)
    out_ref[...] = conf_loss + loc_loss


def kernel(predicted_locs, predicted_scores, boxes, labels, priors):
    B, P, _ = predicted_locs.shape
    C = predicted_scores.shape[2]
    O = boxes.shape[1]

    locs_t = jnp.swapaxes(predicted_locs, 1, 2)
    scores_t = jnp.swapaxes(predicted_scores, 1, 2)
    priors_t = priors.T
    labels_f = labels.astype(jnp.float32)[:, :, None]

    neg, stats = pl.pallas_call(
        _match_body,
        grid=(B,),
        in_specs=[
            pl.BlockSpec((1, 4, P), lambda b: (b, 0, 0)),
            pl.BlockSpec((1, C, P), lambda b: (b, 0, 0)),
            pl.BlockSpec((1, O, 4), lambda b: (b, 0, 0)),
            pl.BlockSpec((1, O, 1), lambda b: (b, 0, 0)),
            pl.BlockSpec((4, P), lambda b: (0, 0)),
        ],
        out_specs=[
            pl.BlockSpec((1, 1, P), lambda b: (b, 0, 0)),
            pl.BlockSpec((1, 1, 128), lambda b: (b, 0, 0)),
        ],
        out_shape=[
            jax.ShapeDtypeStruct((B, 1, P), jnp.float32),
            jax.ShapeDtypeStruct((B, 1, 128), jnp.float32),
        ],
    )(locs_t, scores_t, boxes, labels_f, priors_t)

    out = pl.pallas_call(
        _select_body,
        in_specs=[
            pl.BlockSpec((B, P), lambda: (0, 0)),
            pl.BlockSpec((B, 128), lambda: (0, 0)),
        ],
        out_specs=pl.BlockSpec((1, 1), lambda: (0, 0)),
        out_shape=jax.ShapeDtypeStruct((1, 1), jnp.float32),
    )(neg.reshape(B, P), stats.reshape(B, 128))
    return out[0, 0]


# speedup vs baseline: 11.1833x; 1.0441x over previous
import jax
import jax.numpy as jnp
from jax import lax
from jax.experimental import pallas as pl

_THRESHOLD = 0.5


def _match_body(locs_ref, scores_ref, boxes_ref, labels_ref, priors_ref,
                neg_ref, stats_ref):
    BB = boxes_ref.shape[0]
    O = boxes_ref.shape[1]
    C = scores_ref.shape[1]
    P = priors_ref.shape[1]

    pcx = priors_ref[0:1, :]
    pcy = priors_ref[1:2, :]
    pw = priors_ref[2:3, :]
    ph = priors_ref[3:4, :]
    px1 = pcx - pw * 0.5
    py1 = pcy - ph * 0.5
    px2 = pcx + pw * 0.5
    py2 = pcy + ph * 0.5

    for bb in range(BB):
        _one_image(bb, locs_ref, scores_ref, boxes_ref, labels_ref,
                   neg_ref, stats_ref, O, C, P,
                   pcx, pcy, pw, ph, px1, py1, px2, py2)


def _one_image(bb, locs_ref, scores_ref, boxes_ref, labels_ref,
               neg_ref, stats_ref, O, C, P,
               pcx, pcy, pw, ph, px1, py1, px2, py2):
    bx1 = boxes_ref[bb, :, 0:1]
    by1 = boxes_ref[bb, :, 1:2]
    bx2 = boxes_ref[bb, :, 2:3]
    by2 = boxes_ref[bb, :, 3:4]

    iw = jnp.maximum(jnp.minimum(bx2, px2) - jnp.maximum(bx1, px1), 0.0)
    ih = jnp.maximum(jnp.minimum(by2, py2) - jnp.maximum(by1, py1), 0.0)
    inter = iw * ih
    area_a = (bx2 - bx1) * (by2 - by1)
    area_b = (px2 - px1) * (py2 - py1)
    ov = inter / (area_a + area_b - inter)

    ofep = jnp.max(ov, axis=0, keepdims=True)
    oidx = lax.broadcasted_iota(jnp.int32, (O, P), 0).astype(jnp.float32)
    obj = jnp.min(jnp.where(ov == ofep, oidx, float(O)), axis=0, keepdims=True)

    rowmax = jnp.max(ov, axis=1, keepdims=True)
    pidx = lax.broadcasted_iota(jnp.int32, (O, P), 1).astype(jnp.float32)
    pfeo = jnp.min(jnp.where(ov == rowmax, pidx, float(P)), axis=1,
                   keepdims=True)
    forced = jnp.where(pidx == pfeo, 1.0, 0.0)
    any_forced = jnp.max(forced, axis=0, keepdims=True)
    last_obj = jnp.max(jnp.where(forced > 0.0, oidx, -1.0), axis=0,
                       keepdims=True)
    obj = jnp.where(any_forced > 0.0, last_obj, obj)
    ofep = jnp.where(any_forced > 0.0, 1.0, ofep)

    oh = oidx == obj

    def gather(src):
        return jnp.sum(jnp.where(oh, src, 0.0), axis=0, keepdims=True)

    gx1 = gather(bx1)
    gy1 = gather(by1)
    gx2 = gather(bx2)
    gy2 = gather(by2)
    glab = gather(labels_ref[bb])

    lbl = jnp.where(ofep < _THRESHOLD, 0.0, glab)
    posf = jnp.where(lbl != 0.0, 1.0, 0.0)

    cx = (gx1 + gx2) / 2.0
    cy = (gy1 + gy2) / 2.0
    w = gx2 - gx1
    h = gy2 - gy1
    tgx = (cx - pcx) / (pw / 10.0)
    tgy = (cy - pcy) / (ph / 10.0)
    tgw = jnp.log(w / pw) * 5.0
    tgh = jnp.log(h / ph) * 5.0

    loc_sum = jnp.float32(0.0)
    for c, t in enumerate((tgx, tgy, tgw, tgh)):
        pd = locs_ref[bb, c:c + 1, :]
        pd = jnp.where(jnp.isnan(pd), 0.0, pd)
        t = jnp.where(jnp.isnan(t), 0.0, t)
        d = pd - t
        ad = jnp.abs(d)
        e = jnp.where(ad < 1.0, 0.5 * d * d, ad - 0.5)
        loc_sum = loc_sum + jnp.sum(jnp.where(posf > 0.0, e, 0.0))

    n_pos = jnp.sum(posf)

    x = scores_ref[bb]
    m = jnp.max(x, axis=0, keepdims=True)
    sumexp = jnp.sum(jnp.exp(x - m), axis=0, keepdims=True)
    lse = m + jnp.log(sumexp)
    cidx = lax.broadcasted_iota(jnp.int32, (C, P), 0).astype(jnp.float32)
    xc = jnp.sum(jnp.where(cidx == lbl, x, 0.0), axis=0, keepdims=True)
    conf_all = lse - xc
    conf_pos = jnp.sum(jnp.where(posf > 0.0, conf_all, 0.0))
    neg_ref[bb] = jnp.where(posf > 0.0, 0.0, conf_all)

    lane = lax.broadcasted_iota(jnp.int32, (1, 128), 1).astype(jnp.float32)
    stats_ref[bb] = jnp.where(lane == 0.0, loc_sum,
                   jnp.where(lane == 1.0, n_pos,
                   jnp.where(lane == 2.0, conf_pos, 0.0)))


def _select_body(neg_ref, stats_ref, out_ref):
    B, P = neg_ref.shape
    x = neg_ref[...]
    st = stats_ref[...]
    loc_sum = st[:, 0:1]
    n_pos = st[:, 1:2]
    conf_pos = st[:, 2:3]
    k = jnp.minimum(n_pos * 3.0, float(P))

    i = lax.bitcast_convert_type(x, jnp.int32)
    y = jnp.where(i < 0, i ^ jnp.int32(0x7FFFFFFF), i)

    p = jnp.full((B, 1), -2147483648, dtype=jnp.int32)
    cnt0 = jnp.sum(jnp.where(y >= 0, 1.0, 0.0), axis=1, keepdims=True)
    p = jnp.where(cnt0 >= k, jnp.int32(0), p)
    for bit in range(30, -1, -1):
        cand = p + jnp.int32(1 << bit)
        cnt = jnp.sum(jnp.where(y >= cand, 1.0, 0.0), axis=1, keepdims=True)
        p = jnp.where(cnt >= k, cand, p)

    gt = y > p
    cnt_gt = jnp.sum(jnp.where(gt, 1.0, 0.0), axis=1, keepdims=True)
    sum_gt = jnp.sum(jnp.where(gt, x, 0.0), axis=1, keepdims=True)
    it = jnp.where(p < 0, p ^ jnp.int32(0x7FFFFFFF), p)
    t = lax.bitcast_convert_type(it, jnp.float32)
    rem = k - cnt_gt
    hard = sum_gt + jnp.where(rem > 0.0, rem * t, 0.0)

    npos_tot = jnp.sum(n_pos, keepdims=True)
    conf_loss = (jnp.sum(hard, keepdims=True)
                 + jnp.sum(conf_pos, keepdims=True)) / npos_tot
    loc_loss = jnp.sum(loc_sum, keepdims=True) / (npos_tot * 4.0)
    out_ref[...] = conf_loss + loc_loss


def kernel(predicted_locs, predicted_scores, boxes, labels, priors):
    B, P, _ = predicted_locs.shape
    C = predicted_scores.shape[2]
    O = boxes.shape[1]

    locs_t = jnp.swapaxes(predicted_locs, 1, 2)
    scores_t = jnp.swapaxes(predicted_scores, 1, 2)
    priors_t = priors.T
    labels_f = labels.astype(jnp.float32)[:, :, None]

    BB = 4 if B % 4 == 0 else 1
    from jax.experimental.pallas import tpu as pltpu
    neg, stats = pl.pallas_call(
        _match_body,
        grid=(B // BB,),
        in_specs=[
            pl.BlockSpec((BB, 4, P), lambda b: (b, 0, 0)),
            pl.BlockSpec((BB, C, P), lambda b: (b, 0, 0)),
            pl.BlockSpec((BB, O, 4), lambda b: (b, 0, 0)),
            pl.BlockSpec((BB, O, 1), lambda b: (b, 0, 0)),
            pl.BlockSpec((4, P), lambda b: (0, 0)),
        ],
        out_specs=[
            pl.BlockSpec((BB, 1, P), lambda b: (b, 0, 0)),
            pl.BlockSpec((BB, 1, 128), lambda b: (b, 0, 0)),
        ],
        out_shape=[
            jax.ShapeDtypeStruct((B, 1, P), jnp.float32),
            jax.ShapeDtypeStruct((B, 1, 128), jnp.float32),
        ],
        compiler_params=pltpu.CompilerParams(
            dimension_semantics=("parallel",),
            vmem_limit_bytes=100 * 1024 * 1024,
        ),
    )(locs_t, scores_t, boxes, labels_f, priors_t)

    out = pl.pallas_call(
        _select_body,
        in_specs=[
            pl.BlockSpec((B, P), lambda: (0, 0)),
            pl.BlockSpec((B, 128), lambda: (0, 0)),
        ],
        out_specs=pl.BlockSpec((1, 1), lambda: (0, 0)),
        out_shape=jax.ShapeDtypeStruct((1, 1), jnp.float32),
    )(neg.reshape(B, P), stats.reshape(B, 128))
    return out[0, 0]


# speedup vs baseline: 11.3291x; 1.0130x over previous
import jax
import jax.numpy as jnp
from jax import lax
from jax.experimental import pallas as pl

_THRESHOLD = 0.5


def _match_body(locs_ref, scores_ref, boxes_ref, labels_ref, priors_ref,
                neg_ref, stats_ref):
    BB = boxes_ref.shape[0]
    O = boxes_ref.shape[1]
    C = scores_ref.shape[1]
    P = priors_ref.shape[1]

    pcx = priors_ref[0:1, :]
    pcy = priors_ref[1:2, :]
    pw = priors_ref[2:3, :]
    ph = priors_ref[3:4, :]
    px1 = pcx - pw * 0.5
    py1 = pcy - ph * 0.5
    px2 = pcx + pw * 0.5
    py2 = pcy + ph * 0.5

    for bb in range(BB):
        _one_image(bb, locs_ref, scores_ref, boxes_ref, labels_ref,
                   neg_ref, stats_ref, O, C, P,
                   pcx, pcy, pw, ph, px1, py1, px2, py2)


def _one_image(bb, locs_ref, scores_ref, boxes_ref, labels_ref,
               neg_ref, stats_ref, O, C, P,
               pcx, pcy, pw, ph, px1, py1, px2, py2):
    bx1 = boxes_ref[bb, :, 0:1]
    by1 = boxes_ref[bb, :, 1:2]
    bx2 = boxes_ref[bb, :, 2:3]
    by2 = boxes_ref[bb, :, 3:4]

    iw = jnp.maximum(jnp.minimum(bx2, px2) - jnp.maximum(bx1, px1), 0.0)
    ih = jnp.maximum(jnp.minimum(by2, py2) - jnp.maximum(by1, py1), 0.0)
    inter = iw * ih
    area_a = (bx2 - bx1) * (by2 - by1)
    area_b = (px2 - px1) * (py2 - py1)
    ov = inter / (area_a + area_b - inter)

    ofep = jnp.max(ov, axis=0, keepdims=True)
    oidx = lax.broadcasted_iota(jnp.int32, (O, P), 0).astype(jnp.float32)
    obj = jnp.min(jnp.where(ov == ofep, oidx, float(O)), axis=0, keepdims=True)

    rowmax = jnp.max(ov, axis=1, keepdims=True)
    pidx = lax.broadcasted_iota(jnp.int32, (O, P), 1).astype(jnp.float32)
    pfeo = jnp.min(jnp.where(ov == rowmax, pidx, float(P)), axis=1,
                   keepdims=True)
    forced = jnp.where(pidx == pfeo, 1.0, 0.0)
    any_forced = jnp.max(forced, axis=0, keepdims=True)
    last_obj = jnp.max(jnp.where(forced > 0.0, oidx, -1.0), axis=0,
                       keepdims=True)
    obj = jnp.where(any_forced > 0.0, last_obj, obj)
    ofep = jnp.where(any_forced > 0.0, 1.0, ofep)

    oh = oidx == obj

    def gather(src):
        return jnp.sum(jnp.where(oh, src, 0.0), axis=0, keepdims=True)

    gx1 = gather(bx1)
    gy1 = gather(by1)
    gx2 = gather(bx2)
    gy2 = gather(by2)
    glab = gather(labels_ref[bb])

    lbl = jnp.where(ofep < _THRESHOLD, 0.0, glab)
    posf = jnp.where(lbl != 0.0, 1.0, 0.0)

    cx = (gx1 + gx2) / 2.0
    cy = (gy1 + gy2) / 2.0
    w = gx2 - gx1
    h = gy2 - gy1
    tgx = (cx - pcx) / (pw / 10.0)
    tgy = (cy - pcy) / (ph / 10.0)
    tgw = jnp.log(w / pw) * 5.0
    tgh = jnp.log(h / ph) * 5.0

    loc_sum = jnp.float32(0.0)
    for c, t in enumerate((tgx, tgy, tgw, tgh)):
        pd = locs_ref[bb, c:c + 1, :]
        pd = jnp.where(jnp.isnan(pd), 0.0, pd)
        t = jnp.where(jnp.isnan(t), 0.0, t)
        d = pd - t
        ad = jnp.abs(d)
        e = jnp.where(ad < 1.0, 0.5 * d * d, ad - 0.5)
        loc_sum = loc_sum + jnp.sum(jnp.where(posf > 0.0, e, 0.0))

    n_pos = jnp.sum(posf)

    x = scores_ref[bb]
    m = jnp.max(x, axis=0, keepdims=True)
    sumexp = jnp.sum(jnp.exp(x - m), axis=0, keepdims=True)
    lse = m + jnp.log(sumexp)
    cidx = lax.broadcasted_iota(jnp.int32, (C, P), 0).astype(jnp.float32)
    xc = jnp.sum(jnp.where(cidx == lbl, x, 0.0), axis=0, keepdims=True)
    conf_all = lse - xc
    conf_pos = jnp.sum(jnp.where(posf > 0.0, conf_all, 0.0))
    neg_ref[bb] = jnp.where(posf > 0.0, 0.0, conf_all)

    lane = lax.broadcasted_iota(jnp.int32, (1, 128), 1).astype(jnp.float32)
    stats_ref[bb] = jnp.where(lane == 0.0, loc_sum,
                   jnp.where(lane == 1.0, n_pos,
                   jnp.where(lane == 2.0, conf_pos, 0.0)))


def _select_body(neg_ref, stats_ref, out_ref):
    B, P = neg_ref.shape
    x = neg_ref[...]
    st = stats_ref[...]
    loc_sum = st[:, 0:1]
    n_pos = st[:, 1:2]
    conf_pos = st[:, 2:3]
    k = jnp.minimum(n_pos * 3.0, float(P))

    i = lax.bitcast_convert_type(x, jnp.int32)
    y = jnp.where(i < 0, i ^ jnp.int32(0x7FFFFFFF), i)

    p = jnp.full((B, 1), -2147483648, dtype=jnp.int32)
    cnt0 = jnp.sum(jnp.where(y >= 0, 1.0, 0.0), axis=1, keepdims=True)
    p = jnp.where(cnt0 >= k, jnp.int32(0), p)
    for bit in range(30, -1, -1):
        cand = p + jnp.int32(1 << bit)
        cnt = jnp.sum(jnp.where(y >= cand, 1.0, 0.0), axis=1, keepdims=True)
        p = jnp.where(cnt >= k, cand, p)

    gt = y > p
    cnt_gt = jnp.sum(jnp.where(gt, 1.0, 0.0), axis=1, keepdims=True)
    sum_gt = jnp.sum(jnp.where(gt, x, 0.0), axis=1, keepdims=True)
    it = jnp.where(p < 0, p ^ jnp.int32(0x7FFFFFFF), p)
    t = lax.bitcast_convert_type(it, jnp.float32)
    rem = k - cnt_gt
    hard = sum_gt + jnp.where(rem > 0.0, rem * t, 0.0)

    npos_tot = jnp.sum(n_pos, keepdims=True)
    conf_loss = (jnp.sum(hard, keepdims=True)
                 + jnp.sum(conf_pos, keepdims=True)) / npos_tot
    loc_loss = jnp.sum(loc_sum, keepdims=True) / (npos_tot * 4.0)
    out_ref[...] = conf_loss + loc_loss


def kernel(predicted_locs, predicted_scores, boxes, labels, priors):
    B, P, _ = predicted_locs.shape
    C = predicted_scores.shape[2]
    O = boxes.shape[1]

    locs_t = jnp.swapaxes(predicted_locs, 1, 2)
    scores_t = jnp.swapaxes(predicted_scores, 1, 2)
    priors_t = priors.T
    labels_f = labels.astype(jnp.float32)[:, :, None]

    BB = 4 if B % 4 == 0 else 1
    from jax.experimental.pallas import tpu as pltpu
    neg, stats = pl.pallas_call(
        _match_body,
        grid=(B // BB,),
        in_specs=[
            pl.BlockSpec((BB, 4, P), lambda b: (b, 0, 0)),
            pl.BlockSpec((BB, C, P), lambda b: (b, 0, 0)),
            pl.BlockSpec((BB, O, 4), lambda b: (b, 0, 0)),
            pl.BlockSpec((BB, O, 1), lambda b: (b, 0, 0)),
            pl.BlockSpec((4, P), lambda b: (0, 0)),
        ],
        out_specs=[
            pl.BlockSpec((BB, 1, P), lambda b: (b, 0, 0)),
            pl.BlockSpec((BB, 1, 128), lambda b: (b, 0, 0)),
        ],
        out_shape=[
            jax.ShapeDtypeStruct((B, 1, P), jnp.float32),
            jax.ShapeDtypeStruct((B, 1, 128), jnp.float32),
        ],
        compiler_params=pltpu.CompilerParams(
            dimension_semantics=("parallel",),
            vmem_limit_bytes=100 * 1024 * 1024,
            allow_input_fusion=[True, True, False, False, False],
        ),
    )(locs_t, scores_t, boxes, labels_f, priors_t)

    out = pl.pallas_call(
        _select_body,
        in_specs=[
            pl.BlockSpec((B, P), lambda: (0, 0)),
            pl.BlockSpec((B, 128), lambda: (0, 0)),
        ],
        out_specs=pl.BlockSpec((1, 1), lambda: (0, 0)),
        out_shape=jax.ShapeDtypeStruct((1, 1), jnp.float32),
    )(neg.reshape(B, P), stats.reshape(B, 128))
    return out[0, 0]


# speedup vs baseline: 15.2904x; 1.3497x over previous
import jax
import jax.numpy as jnp
from jax import lax
from jax.experimental import pallas as pl
from jax.experimental.pallas import tpu as pltpu

_THRESHOLD = 0.5
_S = 8


def _match_body(locs_ref, scores_ref, boxes_ref, labels_ref, priors_ref,
                neg_ref, stats_ref):
    BB = boxes_ref.shape[0]
    for bb in range(BB):
        _one_image(bb, locs_ref, scores_ref, boxes_ref, labels_ref,
                   priors_ref, neg_ref, stats_ref)


def _one_image(bb, locs_ref, scores_ref, boxes_ref, labels_ref, priors_ref,
               neg_ref, stats_ref):
    O = boxes_ref.shape[1]
    C = scores_ref.shape[1]
    S, L = priors_ref.shape[1], priors_ref.shape[2]

    pcx = priors_ref[0]
    pcy = priors_ref[1]
    pw = priors_ref[2]
    ph = priors_ref[3]
    px1 = pcx - pw * 0.5
    py1 = pcy - ph * 0.5
    px2 = pcx + pw * 0.5
    py2 = pcy + ph * 0.5

    bx1 = boxes_ref[bb, :, 0:1][:, :, None]
    by1 = boxes_ref[bb, :, 1:2][:, :, None]
    bx2 = boxes_ref[bb, :, 2:3][:, :, None]
    by2 = boxes_ref[bb, :, 3:4][:, :, None]

    iw = jnp.maximum(jnp.minimum(bx2, px2) - jnp.maximum(bx1, px1), 0.0)
    ih = jnp.maximum(jnp.minimum(by2, py2) - jnp.maximum(by1, py1), 0.0)
    inter = iw * ih
    area_a = (bx2 - bx1) * (by2 - by1)
    area_b = (px2 - px1) * (py2 - py1)
    ov = inter / (area_a + area_b - inter)

    ofep = jnp.max(ov, axis=0)
    oidx = lax.broadcasted_iota(jnp.int32, (O, S, L), 0).astype(jnp.float32)
    obj = jnp.min(jnp.where(ov == ofep, oidx, float(O)), axis=0)

    rowmax = jnp.max(ov, axis=(1, 2), keepdims=True)
    pidx = (lax.broadcasted_iota(jnp.int32, (O, S, L), 1) * L
            + lax.broadcasted_iota(jnp.int32, (O, S, L), 2)
            ).astype(jnp.float32)
    pfeo = jnp.min(jnp.where(ov == rowmax, pidx, float(S * L)), axis=(1, 2),
                   keepdims=True)
    forced = jnp.where(pidx == pfeo, 1.0, 0.0)
    any_forced = jnp.max(forced, axis=0)
    last_obj = jnp.max(jnp.where(forced > 0.0, oidx, -1.0), axis=0)
    obj = jnp.where(any_forced > 0.0, last_obj, obj)
    ofep = jnp.where(any_forced > 0.0, 1.0, ofep)

    oh = oidx == obj

    def gather(src):
        return jnp.sum(jnp.where(oh, src, 0.0), axis=0)

    gx1 = gather(bx1)
    gy1 = gather(by1)
    gx2 = gather(bx2)
    gy2 = gather(by2)
    glab = gather(labels_ref[bb, :, 0:1][:, :, None])

    lbl = jnp.where(ofep < _THRESHOLD, 0.0, glab)
    posf = jnp.where(lbl != 0.0, 1.0, 0.0)

    cx = (gx1 + gx2) / 2.0
    cy = (gy1 + gy2) / 2.0
    w = gx2 - gx1
    h = gy2 - gy1
    tgx = (cx - pcx) / (pw / 10.0)
    tgy = (cy - pcy) / (ph / 10.0)
    tgw = jnp.log(w / pw) * 5.0
    tgh = jnp.log(h / ph) * 5.0

    loc_sum = jnp.float32(0.0)
    for c, t in enumerate((tgx, tgy, tgw, tgh)):
        pd = locs_ref[bb, c]
        pd = jnp.where(jnp.isnan(pd), 0.0, pd)
        t = jnp.where(jnp.isnan(t), 0.0, t)
        d = pd - t
        ad = jnp.abs(d)
        e = jnp.where(ad < 1.0, 0.5 * d * d, ad - 0.5)
        loc_sum = loc_sum + jnp.sum(jnp.where(posf > 0.0, e, 0.0))

    n_pos = jnp.sum(posf)

    x = scores_ref[bb]
    m = jnp.max(x, axis=0)
    sumexp = jnp.sum(jnp.exp(x - m), axis=0)
    lse = m + jnp.log(sumexp)
    cidx = lax.broadcasted_iota(jnp.int32, (C, S, L), 0).astype(jnp.float32)
    xc = jnp.sum(jnp.where(cidx == lbl, x, 0.0), axis=0)
    conf_all = lse - xc
    conf_pos = jnp.sum(jnp.where(posf > 0.0, conf_all, 0.0))
    neg_ref[bb] = jnp.where(posf > 0.0, 0.0, conf_all)

    lane = lax.broadcasted_iota(jnp.int32, (1, 128), 1).astype(jnp.float32)
    stats_ref[bb] = jnp.where(lane == 0.0, loc_sum,
                    jnp.where(lane == 1.0, n_pos,
                    jnp.where(lane == 2.0, conf_pos, 0.0)))


def _select_body(neg_ref, stats_ref, out_ref):
    B, P = neg_ref.shape
    x = neg_ref[...]
    st = stats_ref[...]
    loc_sum = st[:, 0:1]
    n_pos = st[:, 1:2]
    conf_pos = st[:, 2:3]
    k = jnp.minimum(n_pos * 3.0, float(P))

    i = lax.bitcast_convert_type(x, jnp.int32)
    y = jnp.where(i < 0, i ^ jnp.int32(0x7FFFFFFF), i)

    p = jnp.full((B, 1), -2147483648, dtype=jnp.int32)
    cnt0 = jnp.sum(jnp.where(y >= 0, 1.0, 0.0), axis=1, keepdims=True)
    p = jnp.where(cnt0 >= k, jnp.int32(0), p)
    for bit in range(30, -1, -1):
        cand = p + jnp.int32(1 << bit)
        cnt = jnp.sum(jnp.where(y >= cand, 1.0, 0.0), axis=1, keepdims=True)
        p = jnp.where(cnt >= k, cand, p)

    gt = y > p
    cnt_gt = jnp.sum(jnp.where(gt, 1.0, 0.0), axis=1, keepdims=True)
    sum_gt = jnp.sum(jnp.where(gt, x, 0.0), axis=1, keepdims=True)
    it = jnp.where(p < 0, p ^ jnp.int32(0x7FFFFFFF), p)
    t = lax.bitcast_convert_type(it, jnp.float32)
    rem = k - cnt_gt
    hard = sum_gt + jnp.where(rem > 0.0, rem * t, 0.0)

    npos_tot = jnp.sum(n_pos, keepdims=True)
    conf_loss = (jnp.sum(hard, keepdims=True)
                 + jnp.sum(conf_pos, keepdims=True)) / npos_tot
    loc_loss = jnp.sum(loc_sum, keepdims=True) / (npos_tot * 4.0)
    out_ref[...] = conf_loss + loc_loss


def kernel(predicted_locs, predicted_scores, boxes, labels, priors):
    B, P, _ = predicted_locs.shape
    C = predicted_scores.shape[2]
    O = boxes.shape[1]
    S = _S if P % _S == 0 else 1
    L = P // S

    locs_t = jnp.swapaxes(predicted_locs, 1, 2).reshape(B, 4, S, L)
    scores_t = jnp.swapaxes(predicted_scores, 1, 2).reshape(B, C, S, L)
    priors_t = priors.T.reshape(4, S, L)
    labels_f = labels.astype(jnp.float32)[:, :, None]

    BB = 4 if B % 4 == 0 else 1
    neg, stats = pl.pallas_call(
        _match_body,
        grid=(B // BB,),
        in_specs=[
            pl.BlockSpec((BB, 4, S, L), lambda b: (b, 0, 0, 0)),
            pl.BlockSpec((BB, C, S, L), lambda b: (b, 0, 0, 0)),
            pl.BlockSpec((BB, O, 4), lambda b: (b, 0, 0)),
            pl.BlockSpec((BB, O, 1), lambda b: (b, 0, 0)),
            pl.BlockSpec((4, S, L), lambda b: (0, 0, 0)),
        ],
        out_specs=[
            pl.BlockSpec((BB, S, L), lambda b: (b, 0, 0)),
            pl.BlockSpec((BB, 1, 128), lambda b: (b, 0, 0)),
        ],
        out_shape=[
            jax.ShapeDtypeStruct((B, S, L), jnp.float32),
            jax.ShapeDtypeStruct((B, 1, 128), jnp.float32),
        ],
        compiler_params=pltpu.CompilerParams(
            dimension_semantics=("parallel",),
            vmem_limit_bytes=100 * 1024 * 1024,
            allow_input_fusion=[True, True, False, False, False],
        ),
    )(locs_t, scores_t, boxes, labels_f, priors_t)

    out = pl.pallas_call(
        _select_body,
        in_specs=[
            pl.BlockSpec((B, P), lambda: (0, 0)),
            pl.BlockSpec((B, 128), lambda: (0, 0)),
        ],
        out_specs=pl.BlockSpec((1, 1), lambda: (0, 0)),
        out_shape=jax.ShapeDtypeStruct((1, 1), jnp.float32),
    )(neg.reshape(B, P), stats.reshape(B, 128))
    return out[0, 0]
